# Initial kernel scaffold; baseline (speedup 1.0000x reference)
import jax
import jax.numpy as jnp
from jax.experimental import pallas as pl


def kernel(x, x_e, edge_index, in_norm_g, in_norm_b, in_proj_W, in_proj_b, e_norm_g, e_norm_b, e_proj_W, e_proj_b, gn0_w, gn0_b, gn0_ms, hg0_W, hg0_b, skip0_W, skip0_b, gnd0_w, gnd0_b, gnd0_ms, hgd0_W, hgd0_b, skipd0_W, skipd0_b, fuse_W, fuse_b, lin_W, lin_b):
    raise NotImplementedError("write your pallas kernel here")


# speedup vs baseline: 1.8735x; 1.8735x over previous
import functools

import jax
import jax.numpy as jnp
from jax import lax
from jax.experimental import pallas as pl
from jax.experimental.pallas import tpu as pltpu

N = 10000
M = 10000
E = 320000
D = 128


def _leaky(x):
    return jnp.where(x >= 0, x, 0.01 * x)


def _matT(a, w):
    return lax.dot_general(a, w, (((1,), (1,)), ((), ())),
                           preferred_element_type=jnp.float32)


def _dense_pre_body(x_ref, xe_ref, ing, inb, ipW, ipb, eng, enb, epW, epb,
                    gn0w, gn0b, gn0ms, hgW, skW, skb,
                    gndw, gndb, gndms, hgdW, skdW, skdb,
                    xw_ref, skipx_ref, xwe_ref, skipxe_ref):
    eps = 1e-5
    x = x_ref[...]
    m = jnp.mean(x, axis=1, keepdims=True)
    v = jnp.mean((x - m) ** 2, axis=1, keepdims=True)
    xln = (x - m) / jnp.sqrt(v + eps) * ing[...] + inb[...]
    x1 = _leaky(_matT(xln, ipW[...]) + ipb[...])
    mu = jnp.mean(x1, axis=0, keepdims=True)
    out = x1 - mu * gn0ms[...]
    var = jnp.mean(out * out, axis=0, keepdims=True)
    xg = gn0w[...] * out / jnp.sqrt(var + eps) + gn0b[...]
    xw_ref[...] = _matT(xg, hgW[...])
    skipx_ref[...] = _matT(xg, skW[...]) + skb[...]
    xe = xe_ref[...]
    nrm = jnp.sqrt(jnp.sum(xe * xe, axis=1, keepdims=True))
    xe = xe / jnp.maximum(nrm, 1e-12)
    me = jnp.mean(xe, axis=1, keepdims=True)
    ve = jnp.mean((xe - me) ** 2, axis=1, keepdims=True)
    xeln = (xe - me) / jnp.sqrt(ve + eps) * eng[...] + enb[...]
    xe1 = _leaky(_matT(xeln, epW[...]) + epb[...])
    mue = jnp.mean(xe1, axis=0, keepdims=True)
    oute = xe1 - mue * gndms[...]
    vare = jnp.mean(oute * oute, axis=0, keepdims=True)
    xeg = gndw[...] * oute / jnp.sqrt(vare + eps) + gndb[...]
    xwe_ref[...] = _matT(xeg, hgdW[...])
    skipxe_ref[...] = _matT(xeg, skdW[...]) + skdb[...]


@jax.jit
def _dense_pre(x, x_e, ing, inb, ipW, ipb, eng, enb, epW, epb,
               gn0w, gn0b, gn0ms, hgW, skW, skb,
               gndw, gndb, gndms, hgdW, skdW, skdb):
    outs = (jax.ShapeDtypeStruct((N, D), jnp.float32),) * 4
    return pl.pallas_call(
        _dense_pre_body,
        out_shape=outs,
    )(x, x_e, ing, inb, ipW, ipb, eng, enb, epW, epb,
      gn0w, gn0b, gn0ms, hgW, skW, skb,
      gndw, gndb, gndms, hgdW, skdW, skdb)


def _mid_body(p1a, p1b, p3a, p3b, cr, cc, oute_ref, oute2_ref, becol_ref, berow_ref):
    cnt_r = cr[...]
    cnt_c = cc[...]
    inv_r = jnp.where(cnt_r > 0, 1.0 / jnp.maximum(cnt_r, 1.0), 0.0)
    inv_c = jnp.where(cnt_c > 0, 1.0 / jnp.maximum(cnt_c, 1.0), 0.0)
    oute_ref[...] = (p1a[...] + p1b[...]) * inv_c.reshape(M, 1)
    oute2_ref[...] = (p3a[...] + p3b[...]) * inv_r.reshape(N, 1)
    becol_ref[...] = inv_c
    berow_ref[...] = inv_r


@jax.jit
def _mid(p1a, p1b, p3a, p3b, cnt_row, cnt_col):
    outs = (jax.ShapeDtypeStruct((M, D), jnp.float32),
            jax.ShapeDtypeStruct((N, D), jnp.float32),
            jax.ShapeDtypeStruct((1, M), jnp.float32),
            jax.ShapeDtypeStruct((1, N), jnp.float32))
    return pl.pallas_call(_mid_body, out_shape=outs)(
        p1a, p1b, p3a, p3b, cnt_row, cnt_col)


def _x2_body(p2a, p2b, dninv, hgb, skipx, x2_ref):
    raw = (p2a[...] + p2b[...]) * dninv[...].reshape(N, 1)
    x2_ref[...] = _leaky(raw + hgb[...]) + skipx[...]


@jax.jit
def _x2(p2a, p2b, dninv, hgb, skipx):
    return pl.pallas_call(
        _x2_body, out_shape=jax.ShapeDtypeStruct((N, D), jnp.float32))(
        p2a, p2b, dninv, hgb, skipx)


def _final_body(p4a, p4b, dninv_c, hgdb, skipxe, agg, fuseW, fuseb,
                linW, linb, out_ref):
    raw = (p4a[...] + p4b[...]) * dninv_c[...].reshape(M, 1)
    xe2 = _leaky(raw + hgdb[...]) + skipxe[...]
    fW = fuseW[...]
    h = (_matT(agg[...], fW[:, :D]) + _matT(xe2, fW[:, D:]) + fuseb[...])
    out_ref[...] = _matT(h, linW[...]) + linb[...]


@jax.jit
def _final(p4a, p4b, dninv_c, hgdb, skipxe, agg, fuseW, fuseb, linW, linb):
    return pl.pallas_call(
        _final_body, out_shape=jax.ShapeDtypeStruct((M, D), jnp.float32))(
        p4a, p4b, dninv_c, hgdb, skipxe, agg, fuseW, fuseb, linW, linb)


def kernel(x, x_e, edge_index, in_norm_g, in_norm_b, in_proj_W, in_proj_b,
           e_norm_g, e_norm_b, e_proj_W, e_proj_b, gn0_w, gn0_b, gn0_ms,
           hg0_W, hg0_b, skip0_W, skip0_b, gnd0_w, gnd0_b, gnd0_ms,
           hgd0_W, hgd0_b, skipd0_W, skipd0_b, fuse_W, fuse_b, lin_W, lin_b):
    r2 = lambda a: a.reshape(1, D)
    row, col = edge_index[0], edge_index[1]

    xw, skip_x, xwe, skip_xe = _dense_pre(
        x, x_e, r2(in_norm_g), r2(in_norm_b), in_proj_W, r2(in_proj_b),
        r2(e_norm_g), r2(e_norm_b), e_proj_W, r2(e_proj_b),
        r2(gn0_w), r2(gn0_b), r2(gn0_ms), hg0_W, skip0_W, r2(skip0_b),
        r2(gnd0_w), r2(gnd0_b), r2(gnd0_ms), hgd0_W, skipd0_W, r2(skipd0_b))

    ones_e = jnp.ones((E,), jnp.float32)
    cnt_row = jax.ops.segment_sum(ones_e, row, num_segments=N).reshape(1, N)
    cnt_col = jax.ops.segment_sum(ones_e, col, num_segments=M).reshape(1, M)
    p1 = jax.ops.segment_sum(xw[row], col, num_segments=M)
    p3 = jax.ops.segment_sum(xwe[col], row, num_segments=N)
    zM = jnp.zeros((M, D), jnp.float32)
    zN = jnp.zeros((N, D), jnp.float32)

    out_e, out_e2, inv_c, inv_r = _mid(p1, zM, p3, zN, cnt_row, cnt_col)

    p2 = jax.ops.segment_sum(out_e[col], row, num_segments=N)
    x2 = _x2(p2, zN, inv_r, r2(hg0_b), skip_x)

    p4 = jax.ops.segment_sum(out_e2[row], col, num_segments=M)
    agg = jax.ops.segment_min(x2[row], col, num_segments=M)

    return _final(p4, zM, inv_c, r2(hgd0_b), skip_xe,
                  agg, fuse_W, r2(fuse_b), lin_W, r2(lin_b))


# speedup vs baseline: 4.1435x; 2.2117x over previous
import jax
import jax.numpy as jnp
from jax import lax
from jax.experimental import pallas as pl
from jax.experimental.pallas import tpu as pltpu
from jax.experimental.pallas import tpu_sc as plsc

N = 10000
M = 10000
E = 320000
D = 128

NC, NS = 2, 16
NW = NC * NS
BLK = 128
NBLK = E // BLK
BASE_PW = NBLK // NW
EXTRA = NBLK - BASE_PW * NW
NPAD = 10240
RPS = NPAD // NS

def _leaky(x):
    return jnp.where(x >= 0, x, 0.01 * x)


def _matT(a, w):
    return lax.dot_general(a, w, (((1,), (1,)), ((), ())),
                           preferred_element_type=jnp.float32)


def _sc_pass_body(gidx_hbm, sidx_hbm, table_hbm, out_hbm,
                 acc_sh, rows_v, gi_v, si_v, zb_v, sem):
    cid = lax.axis_index("c")
    sid = lax.axis_index("s")
    wid = cid * NS + sid
    zeros16 = jnp.zeros((16,), jnp.float32)

    def zb_row(r, c):
        for j in range(D // 16):
            zb_v[r, pl.ds(j * 16, 16)] = zeros16
        return c
    lax.fori_loop(0, 32, zb_row, 0)

    def zacc(i, c):
        pltpu.sync_copy(zb_v, acc_sh.at[pl.ds(sid * RPS + i * 32, 32)])
        return c
    lax.fori_loop(0, RPS // 32, zacc, 0)

    plsc.subcore_barrier()

    start = wid * BASE_PW + jnp.minimum(wid, EXTRA)
    nblk = BASE_PW + jnp.where(wid < EXTRA, 1, 0)

    def blk_body(i, c):
        b = start + i
        pltpu.sync_copy(gidx_hbm.at[pl.ds(b, 1)], gi_v)
        pltpu.sync_copy(sidx_hbm.at[pl.ds(b, 1)], si_v)
        pltpu.async_copy(table_hbm.at[gi_v.at[0]], rows_v, sem).wait()
        pltpu.sync_copy(rows_v, acc_sh.at[si_v.at[0]], add=True)
        return c
    lax.fori_loop(0, nblk, blk_body, 0)

    plsc.subcore_barrier()
    pltpu.sync_copy(acc_sh.at[pl.ds(sid * RPS, RPS)],
                    out_hbm.at[cid, pl.ds(sid * RPS, RPS)])


_sc_pass = pl.kernel(
    _sc_pass_body,
    out_type=jax.ShapeDtypeStruct((NC, NPAD, D), jnp.float32),
    mesh=plsc.VectorSubcoreMesh(core_axis_name="c", subcore_axis_name="s"),
    scratch_types=[
        pltpu.VMEM_SHARED((NPAD, D), jnp.float32),
        pltpu.VMEM((BLK, D), jnp.float32),
        pltpu.VMEM((1, BLK), jnp.int32),
        pltpu.VMEM((1, BLK), jnp.int32),
        pltpu.VMEM((32, D), jnp.float32),
        pltpu.SemaphoreType.DMA,
    ])


def _sc_count_body(gidx_hbm, sidx_hbm, out_hbm,
                   acc_sh, onesA_v, onesB_v, gi_v, si_v, zb_v, sem):
    cid = lax.axis_index("c")
    sid = lax.axis_index("s")
    wid = cid * NS + sid
    zeros16 = jnp.zeros((16,), jnp.float32)
    ones16 = jnp.ones((16,), jnp.float32)

    def ofill(r, c):
        for j in range(D // 16):
            onesA_v[r, pl.ds(j * 16, 16)] = ones16 if j == 0 else zeros16
            onesB_v[r, pl.ds(j * 16, 16)] = ones16 if j == 1 else zeros16
            zb_v[r % 32, pl.ds(j * 16, 16)] = zeros16
        return c
    lax.fori_loop(0, BLK, ofill, 0)

    def zacc(i, c):
        pltpu.sync_copy(zb_v, acc_sh.at[pl.ds(sid * RPS + i * 32, 32)])
        return c
    lax.fori_loop(0, RPS // 32, zacc, 0)

    plsc.subcore_barrier()

    start = wid * BASE_PW + jnp.minimum(wid, EXTRA)
    nblk = BASE_PW + jnp.where(wid < EXTRA, 1, 0)

    def blk_body(i, c):
        b = start + i
        pltpu.sync_copy(gidx_hbm.at[pl.ds(b, 1)], gi_v)
        pltpu.sync_copy(sidx_hbm.at[pl.ds(b, 1)], si_v)
        pltpu.sync_copy(onesA_v, acc_sh.at[gi_v.at[0]], add=True)
        pltpu.sync_copy(onesB_v, acc_sh.at[si_v.at[0]], add=True)
        return c
    lax.fori_loop(0, nblk, blk_body, 0)

    plsc.subcore_barrier()
    pltpu.sync_copy(acc_sh.at[pl.ds(sid * RPS, RPS)],
                    out_hbm.at[cid, pl.ds(sid * RPS, RPS)])


_sc_count = pl.kernel(
    _sc_count_body,
    out_type=jax.ShapeDtypeStruct((NC, NPAD, D), jnp.float32),
    mesh=plsc.VectorSubcoreMesh(core_axis_name="c", subcore_axis_name="s"),
    scratch_types=[
        pltpu.VMEM_SHARED((NPAD, D), jnp.float32),
        pltpu.VMEM((BLK, D), jnp.float32),
        pltpu.VMEM((BLK, D), jnp.float32),
        pltpu.VMEM((1, BLK), jnp.int32),
        pltpu.VMEM((1, BLK), jnp.int32),
        pltpu.VMEM((32, D), jnp.float32),
        pltpu.SemaphoreType.DMA,
    ])


MINBLK = 128
MIN_GRID = E // MINBLK
NACC = 4


def _tc_min_body(col_ref, row_ref, x2_ref, out_ref, acc_v):
    step = pl.program_id(0)

    @pl.when(step == 0)
    def _init():
        acc_v[...] = jnp.full((NACC, M, D), jnp.inf, jnp.float32)

    for i in range(0, MINBLK, NACC):
        for k in range(NACC):
            c = col_ref[0, 0, i + k]
            r = row_ref[0, 0, i + k]
            acc_v[k, pl.ds(c, 1), :] = jnp.minimum(
                acc_v[k, pl.ds(c, 1), :], x2_ref[pl.ds(r, 1), :])

    @pl.when(step == MIN_GRID - 1)
    def _fin():
        red = jnp.minimum(
            jnp.minimum(acc_v[0], acc_v[1]),
            jnp.minimum(acc_v[2], acc_v[3]))
        out_ref[...] = red


def _tc_min(col2, row2, x2):
    col3 = col2.reshape(NBLK, 1, BLK)
    row3 = row2.reshape(NBLK, 1, BLK)
    return pl.pallas_call(
        _tc_min_body,
        grid=(MIN_GRID,),
        in_specs=[
            pl.BlockSpec((1, 1, MINBLK), lambda i: (i, 0, 0),
                         memory_space=pltpu.SMEM),
            pl.BlockSpec((1, 1, MINBLK), lambda i: (i, 0, 0),
                         memory_space=pltpu.SMEM),
            pl.BlockSpec((N, D), lambda i: (0, 0)),
        ],
        out_specs=pl.BlockSpec((M, D), lambda i: (0, 0)),
        out_shape=jax.ShapeDtypeStruct((M, D), jnp.float32),
        scratch_shapes=[pltpu.VMEM((NACC, M, D), jnp.float32)],
    )(col3, row3, x2)


def _dense_pre_body(x_ref, xe_ref, ing, inb, ipW, ipb, eng, enb, epW, epb,
                    gn0w, gn0b, gn0ms, hgW, skW, skb,
                    gndw, gndb, gndms, hgdW, skdW, skdb,
                    xw_ref, skipx_ref, xwe_ref, skipxe_ref):
    eps = 1e-5
    x = x_ref[...]
    m = jnp.mean(x, axis=1, keepdims=True)
    v = jnp.mean((x - m) ** 2, axis=1, keepdims=True)
    xln = (x - m) / jnp.sqrt(v + eps) * ing[...] + inb[...]
    x1 = _leaky(_matT(xln, ipW[...]) + ipb[...])
    mu = jnp.mean(x1, axis=0, keepdims=True)
    out = x1 - mu * gn0ms[...]
    var = jnp.mean(out * out, axis=0, keepdims=True)
    xg = gn0w[...] * out / jnp.sqrt(var + eps) + gn0b[...]
    xw_ref[...] = _matT(xg, hgW[...])
    skipx_ref[...] = _matT(xg, skW[...]) + skb[...]
    xe = xe_ref[...]
    nrm = jnp.sqrt(jnp.sum(xe * xe, axis=1, keepdims=True))
    xe = xe / jnp.maximum(nrm, 1e-12)
    me = jnp.mean(xe, axis=1, keepdims=True)
    ve = jnp.mean((xe - me) ** 2, axis=1, keepdims=True)
    xeln = (xe - me) / jnp.sqrt(ve + eps) * eng[...] + enb[...]
    xe1 = _leaky(_matT(xeln, epW[...]) + epb[...])
    mue = jnp.mean(xe1, axis=0, keepdims=True)
    oute = xe1 - mue * gndms[...]
    vare = jnp.mean(oute * oute, axis=0, keepdims=True)
    xeg = gndw[...] * oute / jnp.sqrt(vare + eps) + gndb[...]
    xwe_ref[...] = _matT(xeg, hgdW[...])
    skipxe_ref[...] = _matT(xeg, skdW[...]) + skdb[...]


def _dense_pre(x, x_e, ing, inb, ipW, ipb, eng, enb, epW, epb,
               gn0w, gn0b, gn0ms, hgW, skW, skb,
               gndw, gndb, gndms, hgdW, skdW, skdb):
    outs = (jax.ShapeDtypeStruct((N, D), jnp.float32),) * 4
    return pl.pallas_call(
        _dense_pre_body,
        out_shape=outs,
    )(x, x_e, ing, inb, ipW, ipb, eng, enb, epW, epb,
      gn0w, gn0b, gn0ms, hgW, skW, skb,
      gndw, gndb, gndms, hgdW, skdW, skdb)


def _mid_body(p1, p3, cr, cc, oute_ref, oute2_ref,
              becol_ref, berow_ref):
    cnt_r = (cr[0] + cr[1]).reshape(1, N)
    cnt_c = (cc[0] + cc[1]).reshape(1, M)
    inv_r = jnp.where(cnt_r > 0, 1.0 / jnp.maximum(cnt_r, 1.0), 0.0)
    inv_c = jnp.where(cnt_c > 0, 1.0 / jnp.maximum(cnt_c, 1.0), 0.0)
    oute_ref[...] = (p1[0, :M, :] + p1[1, :M, :]) * inv_c.reshape(M, 1)
    oute2_ref[...] = (p3[0, :N, :] + p3[1, :N, :]) * inv_r.reshape(N, 1)
    becol_ref[...] = inv_c
    berow_ref[...] = inv_r


def _mid(p1, p3, cr, cc):
    outs = (jax.ShapeDtypeStruct((M, D), jnp.float32),
            jax.ShapeDtypeStruct((N, D), jnp.float32),
            jax.ShapeDtypeStruct((1, M), jnp.float32),
            jax.ShapeDtypeStruct((1, N), jnp.float32))
    return pl.pallas_call(_mid_body, out_shape=outs)(p1, p3, cr, cc)


def _x2_body(p2, dninv, hgb, skipx, x2_ref):
    raw = (p2[0, :N, :] + p2[1, :N, :]) * dninv[...].reshape(N, 1)
    x2_ref[...] = _leaky(raw + hgb[...]) + skipx[...]


def _x2(p2, dninv, hgb, skipx):
    return pl.pallas_call(
        _x2_body, out_shape=jax.ShapeDtypeStruct((N, D), jnp.float32))(
        p2, dninv, hgb, skipx)


def _final_body(p4, dninv_c, hgdb, skipxe, agg, fuseW, fuseb,
                linW, linb, out_ref):
    raw = (p4[0, :M, :] + p4[1, :M, :]) * dninv_c[...].reshape(M, 1)
    xe2 = _leaky(raw + hgdb[...]) + skipxe[...]
    fW = fuseW[...]
    h = (_matT(agg[...], fW[:, :D]) + _matT(xe2, fW[:, D:]) + fuseb[...])
    out_ref[...] = _matT(h, linW[...]) + linb[...]


def _final(p4, dninv_c, hgdb, skipxe, agg, fuseW, fuseb, linW, linb):
    return pl.pallas_call(
        _final_body, out_shape=jax.ShapeDtypeStruct((M, D), jnp.float32))(
        p4, dninv_c, hgdb, skipxe, agg, fuseW, fuseb, linW, linb)


def kernel(x, x_e, edge_index, in_norm_g, in_norm_b, in_proj_W, in_proj_b,
           e_norm_g, e_norm_b, e_proj_W, e_proj_b, gn0_w, gn0_b, gn0_ms,
           hg0_W, hg0_b, skip0_W, skip0_b, gnd0_w, gnd0_b, gnd0_ms,
           hgd0_W, hgd0_b, skipd0_W, skipd0_b, fuse_W, fuse_b, lin_W, lin_b):
    r2 = lambda a: a.reshape(1, D)
    row, col = edge_index[0], edge_index[1]
    row2 = row.reshape(NBLK, BLK)
    col2 = col.reshape(NBLK, BLK)

    xw, skip_x, xwe, skip_xe = _dense_pre(
        x, x_e, r2(in_norm_g), r2(in_norm_b), in_proj_W, r2(in_proj_b),
        r2(e_norm_g), r2(e_norm_b), e_proj_W, r2(e_proj_b),
        r2(gn0_w), r2(gn0_b), r2(gn0_ms), hg0_W, skip0_W, r2(skip0_b),
        r2(gnd0_w), r2(gnd0_b), r2(gnd0_ms), hgd0_W, skipd0_W, r2(skipd0_b))

    cntp = _sc_count(row2, col2)
    p1p = _sc_pass(row2, col2, xw)
    p3p = _sc_pass(col2, row2, xwe)
    out_e, out_e2, inv_c, inv_r = _mid(p1p, p3p,
                                       cntp[:, :N, 0], cntp[:, :M, 16])

    p2p = _sc_pass(col2, row2, out_e)
    x2 = _x2(p2p, inv_r, r2(hg0_b), skip_x)

    p4p = _sc_pass(row2, col2, out_e2)
    agg = _tc_min(col2, row2, x2)

    return _final(p4p, inv_c, r2(hgd0_b), skip_xe,
                  agg, fuse_W, r2(fuse_b), lin_W, r2(lin_b))


# speedup vs baseline: 4.6102x; 1.1126x over previous
import jax
import jax.numpy as jnp
from jax import lax
from jax.experimental import pallas as pl
from jax.experimental.pallas import tpu as pltpu
from jax.experimental.pallas import tpu_sc as plsc

N = 10000
M = 10000
E = 320000
D = 128

NC, NS = 2, 16
NW = NC * NS
BLK = 128
NBLK = E // BLK
BASE_PW = NBLK // NW
EXTRA = NBLK - BASE_PW * NW
NPAD = 10240
RPS = NPAD // NS

def _leaky(x):
    return jnp.where(x >= 0, x, 0.01 * x)


def _matT(a, w):
    return lax.dot_general(a, w, (((1,), (1,)), ((), ())),
                           preferred_element_type=jnp.float32)


def _sc_pass_body(idx_hbm, table_hbm, out_hbm,
                  acc_sh, rows0_v, rows1_v, ix0_v, ix1_v, zb_v, sem0, sem1):
    cid = lax.axis_index("c")
    sid = lax.axis_index("s")
    wid = cid * NS + sid
    zeros16 = jnp.zeros((16,), jnp.float32)

    def zb_row(r, c):
        for j in range(D // 16):
            zb_v[r, pl.ds(j * 16, 16)] = zeros16
        return c
    lax.fori_loop(0, 32, zb_row, 0)

    def zacc(i, c):
        pltpu.sync_copy(zb_v, acc_sh.at[pl.ds(sid * RPS + i * 32, 32)])
        return c
    lax.fori_loop(0, RPS // 32, zacc, 0)

    plsc.subcore_barrier()

    start = wid * BASE_PW + jnp.minimum(wid, EXTRA)
    nblk = BASE_PW + jnp.where(wid < EXTRA, 1, 0)

    def pair_body(p, c):
        b0 = start + 2 * p
        pltpu.sync_copy(idx_hbm.at[pl.ds(b0, 1)], ix0_v)
        g0 = pltpu.async_copy(table_hbm.at[ix0_v.at[0, 0]], rows0_v, sem0)
        pltpu.sync_copy(idx_hbm.at[pl.ds(b0 + 1, 1)], ix1_v)
        g1 = pltpu.async_copy(table_hbm.at[ix1_v.at[0, 0]], rows1_v, sem1)
        g0.wait()
        pltpu.sync_copy(rows0_v, acc_sh.at[ix0_v.at[0, 1]], add=True)
        g1.wait()
        pltpu.sync_copy(rows1_v, acc_sh.at[ix1_v.at[0, 1]], add=True)
        return c
    lax.fori_loop(0, nblk // 2, pair_body, 0)

    @pl.when(nblk % 2 == 1)
    def _tail():
        b = start + nblk - 1
        pltpu.sync_copy(idx_hbm.at[pl.ds(b, 1)], ix0_v)
        pltpu.async_copy(table_hbm.at[ix0_v.at[0, 0]], rows0_v, sem0).wait()
        pltpu.sync_copy(rows0_v, acc_sh.at[ix0_v.at[0, 1]], add=True)

    plsc.subcore_barrier()
    pltpu.sync_copy(acc_sh.at[pl.ds(sid * RPS, RPS)],
                    out_hbm.at[cid, pl.ds(sid * RPS, RPS)])


_sc_pass = pl.kernel(
    _sc_pass_body,
    out_type=jax.ShapeDtypeStruct((NC, NPAD, D), jnp.float32),
    mesh=plsc.VectorSubcoreMesh(core_axis_name="c", subcore_axis_name="s"),
    scratch_types=[
        pltpu.VMEM_SHARED((NPAD, D), jnp.float32),
        pltpu.VMEM((BLK, D), jnp.float32),
        pltpu.VMEM((BLK, D), jnp.float32),
        pltpu.VMEM((1, 2, BLK), jnp.int32),
        pltpu.VMEM((1, 2, BLK), jnp.int32),
        pltpu.VMEM((32, D), jnp.float32),
        pltpu.SemaphoreType.DMA,
        pltpu.SemaphoreType.DMA,
    ])


def _sc_count_body(gidx_hbm, sidx_hbm, out_hbm,
                   acc_sh, onesA_v, onesB_v, gi_v, si_v, zb_v, sem):
    cid = lax.axis_index("c")
    sid = lax.axis_index("s")
    wid = cid * NS + sid
    zeros16 = jnp.zeros((16,), jnp.float32)
    ones16 = jnp.ones((16,), jnp.float32)

    def ofill(r, c):
        for j in range(D // 16):
            onesA_v[r, pl.ds(j * 16, 16)] = ones16 if j == 0 else zeros16
            onesB_v[r, pl.ds(j * 16, 16)] = ones16 if j == 1 else zeros16
            zb_v[r % 32, pl.ds(j * 16, 16)] = zeros16
        return c
    lax.fori_loop(0, BLK, ofill, 0)

    def zacc(i, c):
        pltpu.sync_copy(zb_v, acc_sh.at[pl.ds(sid * RPS + i * 32, 32)])
        return c
    lax.fori_loop(0, RPS // 32, zacc, 0)

    plsc.subcore_barrier()

    start = wid * BASE_PW + jnp.minimum(wid, EXTRA)
    nblk = BASE_PW + jnp.where(wid < EXTRA, 1, 0)

    def blk_body(i, c):
        b = start + i
        pltpu.sync_copy(gidx_hbm.at[pl.ds(b, 1)], gi_v)
        pltpu.sync_copy(sidx_hbm.at[pl.ds(b, 1)], si_v)
        pltpu.sync_copy(onesA_v, acc_sh.at[gi_v.at[0]], add=True)
        pltpu.sync_copy(onesB_v, acc_sh.at[si_v.at[0]], add=True)
        return c
    lax.fori_loop(0, nblk, blk_body, 0)

    plsc.subcore_barrier()
    pltpu.sync_copy(acc_sh.at[pl.ds(sid * RPS, RPS)],
                    out_hbm.at[cid, pl.ds(sid * RPS, RPS)])


_sc_count = pl.kernel(
    _sc_count_body,
    out_type=jax.ShapeDtypeStruct((NC, NPAD, D), jnp.float32),
    mesh=plsc.VectorSubcoreMesh(core_axis_name="c", subcore_axis_name="s"),
    scratch_types=[
        pltpu.VMEM_SHARED((NPAD, D), jnp.float32),
        pltpu.VMEM((BLK, D), jnp.float32),
        pltpu.VMEM((BLK, D), jnp.float32),
        pltpu.VMEM((1, BLK), jnp.int32),
        pltpu.VMEM((1, BLK), jnp.int32),
        pltpu.VMEM((32, D), jnp.float32),
        pltpu.SemaphoreType.DMA,
    ])


MINBLK = 128
MIN_GRID = E // MINBLK
NACC = 8


def _tc_min_body(col_ref, row_ref, x2_ref, out_ref, acc_v):
    step = pl.program_id(0)

    @pl.when(step == 0)
    def _init():
        acc_v[...] = jnp.full((NACC, M, D), jnp.inf, jnp.float32)

    for i in range(0, MINBLK, NACC):
        for k in range(NACC):
            c = col_ref[0, 0, i + k]
            r = row_ref[0, 0, i + k]
            acc_v[k, pl.ds(c, 1), :] = jnp.minimum(
                acc_v[k, pl.ds(c, 1), :], x2_ref[pl.ds(r, 1), :])

    @pl.when(step == MIN_GRID - 1)
    def _fin():
        red = acc_v[0]
        for k in range(1, NACC):
            red = jnp.minimum(red, acc_v[k])
        out_ref[...] = red


def _tc_min(col2, row2, x2):
    col3 = col2.reshape(MIN_GRID, 1, MINBLK)
    row3 = row2.reshape(MIN_GRID, 1, MINBLK)
    return pl.pallas_call(
        _tc_min_body,
        grid=(MIN_GRID,),
        in_specs=[
            pl.BlockSpec((1, 1, MINBLK), lambda i: (i, 0, 0),
                         memory_space=pltpu.SMEM),
            pl.BlockSpec((1, 1, MINBLK), lambda i: (i, 0, 0),
                         memory_space=pltpu.SMEM),
            pl.BlockSpec((N, D), lambda i: (0, 0)),
        ],
        out_specs=pl.BlockSpec((M, D), lambda i: (0, 0)),
        out_shape=jax.ShapeDtypeStruct((M, D), jnp.float32),
        scratch_shapes=[pltpu.VMEM((NACC, M, D), jnp.float32)],
    )(col3, row3, x2)


def _dense_pre_body(x_ref, xe_ref, ing, inb, ipW, ipb, eng, enb, epW, epb,
                    gn0w, gn0b, gn0ms, hgW, skW, skb,
                    gndw, gndb, gndms, hgdW, skdW, skdb,
                    xw_ref, skipx_ref, xwe_ref, skipxe_ref):
    eps = 1e-5
    x = x_ref[...]
    m = jnp.mean(x, axis=1, keepdims=True)
    v = jnp.mean((x - m) ** 2, axis=1, keepdims=True)
    xln = (x - m) / jnp.sqrt(v + eps) * ing[...] + inb[...]
    x1 = _leaky(_matT(xln, ipW[...]) + ipb[...])
    mu = jnp.mean(x1, axis=0, keepdims=True)
    out = x1 - mu * gn0ms[...]
    var = jnp.mean(out * out, axis=0, keepdims=True)
    xg = gn0w[...] * out / jnp.sqrt(var + eps) + gn0b[...]
    xw_ref[...] = _matT(xg, hgW[...])
    skipx_ref[...] = _matT(xg, skW[...]) + skb[...]
    xe = xe_ref[...]
    nrm = jnp.sqrt(jnp.sum(xe * xe, axis=1, keepdims=True))
    xe = xe / jnp.maximum(nrm, 1e-12)
    me = jnp.mean(xe, axis=1, keepdims=True)
    ve = jnp.mean((xe - me) ** 2, axis=1, keepdims=True)
    xeln = (xe - me) / jnp.sqrt(ve + eps) * eng[...] + enb[...]
    xe1 = _leaky(_matT(xeln, epW[...]) + epb[...])
    mue = jnp.mean(xe1, axis=0, keepdims=True)
    oute = xe1 - mue * gndms[...]
    vare = jnp.mean(oute * oute, axis=0, keepdims=True)
    xeg = gndw[...] * oute / jnp.sqrt(vare + eps) + gndb[...]
    xwe_ref[...] = _matT(xeg, hgdW[...])
    skipxe_ref[...] = _matT(xeg, skdW[...]) + skdb[...]


def _dense_pre(x, x_e, ing, inb, ipW, ipb, eng, enb, epW, epb,
               gn0w, gn0b, gn0ms, hgW, skW, skb,
               gndw, gndb, gndms, hgdW, skdW, skdb):
    outs = (jax.ShapeDtypeStruct((N, D), jnp.float32),) * 4
    return pl.pallas_call(
        _dense_pre_body,
        out_shape=outs,
    )(x, x_e, ing, inb, ipW, ipb, eng, enb, epW, epb,
      gn0w, gn0b, gn0ms, hgW, skW, skb,
      gndw, gndb, gndms, hgdW, skdW, skdb)


def _mid_body(p1, p3, cr, cc, oute_ref, oute2_ref,
              becol_ref, berow_ref):
    cnt_r = (cr[0] + cr[1]).reshape(1, N)
    cnt_c = (cc[0] + cc[1]).reshape(1, M)
    inv_r = jnp.where(cnt_r > 0, 1.0 / jnp.maximum(cnt_r, 1.0), 0.0)
    inv_c = jnp.where(cnt_c > 0, 1.0 / jnp.maximum(cnt_c, 1.0), 0.0)
    oute_ref[...] = (p1[0, :M, :] + p1[1, :M, :]) * inv_c.reshape(M, 1)
    oute2_ref[...] = (p3[0, :N, :] + p3[1, :N, :]) * inv_r.reshape(N, 1)
    becol_ref[...] = inv_c
    berow_ref[...] = inv_r


def _mid(p1, p3, cr, cc):
    outs = (jax.ShapeDtypeStruct((M, D), jnp.float32),
            jax.ShapeDtypeStruct((N, D), jnp.float32),
            jax.ShapeDtypeStruct((1, M), jnp.float32),
            jax.ShapeDtypeStruct((1, N), jnp.float32))
    return pl.pallas_call(_mid_body, out_shape=outs)(p1, p3, cr, cc)


def _x2_body(p2, dninv, hgb, skipx, x2_ref):
    raw = (p2[0, :N, :] + p2[1, :N, :]) * dninv[...].reshape(N, 1)
    x2_ref[...] = _leaky(raw + hgb[...]) + skipx[...]


def _x2(p2, dninv, hgb, skipx):
    return pl.pallas_call(
        _x2_body, out_shape=jax.ShapeDtypeStruct((N, D), jnp.float32))(
        p2, dninv, hgb, skipx)


def _final_body(p4, dninv_c, hgdb, skipxe, agg, fuseW, fuseb,
                linW, linb, out_ref):
    raw = (p4[0, :M, :] + p4[1, :M, :]) * dninv_c[...].reshape(M, 1)
    xe2 = _leaky(raw + hgdb[...]) + skipxe[...]
    fW = fuseW[...]
    h = (_matT(agg[...], fW[:, :D]) + _matT(xe2, fW[:, D:]) + fuseb[...])
    out_ref[...] = _matT(h, linW[...]) + linb[...]


def _final(p4, dninv_c, hgdb, skipxe, agg, fuseW, fuseb, linW, linb):
    return pl.pallas_call(
        _final_body, out_shape=jax.ShapeDtypeStruct((M, D), jnp.float32))(
        p4, dninv_c, hgdb, skipxe, agg, fuseW, fuseb, linW, linb)


def kernel(x, x_e, edge_index, in_norm_g, in_norm_b, in_proj_W, in_proj_b,
           e_norm_g, e_norm_b, e_proj_W, e_proj_b, gn0_w, gn0_b, gn0_ms,
           hg0_W, hg0_b, skip0_W, skip0_b, gnd0_w, gnd0_b, gnd0_ms,
           hgd0_W, hgd0_b, skipd0_W, skipd0_b, fuse_W, fuse_b, lin_W, lin_b):
    r2 = lambda a: a.reshape(1, D)
    row, col = edge_index[0], edge_index[1]
    row2 = row.reshape(NBLK, BLK)
    col2 = col.reshape(NBLK, BLK)
    rc2 = jnp.stack([row2, col2], axis=1)
    cr2 = jnp.stack([col2, row2], axis=1)

    xw, skip_x, xwe, skip_xe = _dense_pre(
        x, x_e, r2(in_norm_g), r2(in_norm_b), in_proj_W, r2(in_proj_b),
        r2(e_norm_g), r2(e_norm_b), e_proj_W, r2(e_proj_b),
        r2(gn0_w), r2(gn0_b), r2(gn0_ms), hg0_W, skip0_W, r2(skip0_b),
        r2(gnd0_w), r2(gnd0_b), r2(gnd0_ms), hgd0_W, skipd0_W, r2(skipd0_b))

    cntp = _sc_count(row2, col2)
    p1p = _sc_pass(rc2, xw)
    p3p = _sc_pass(cr2, xwe)
    out_e, out_e2, inv_c, inv_r = _mid(p1p, p3p,
                                       cntp[:, :N, 0], cntp[:, :M, 16])

    p2p = _sc_pass(cr2, out_e)
    x2 = _x2(p2p, inv_r, r2(hg0_b), skip_x)

    p4p = _sc_pass(rc2, out_e2)
    agg = _tc_min(col2, row2, x2)

    return _final(p4p, inv_c, r2(hgd0_b), skip_xe,
                  agg, fuse_W, r2(fuse_b), lin_W, r2(lin_b))


# speedup vs baseline: 6.2058x; 1.3461x over previous
import jax
import jax.numpy as jnp
from jax import lax
from jax.experimental import pallas as pl
from jax.experimental.pallas import tpu as pltpu
from jax.experimental.pallas import tpu_sc as plsc

N = 10000
M = 10000
E = 320000
D = 128

NC, NS = 2, 16
NW = NC * NS
BLK = 128
NBLK = E // BLK
BASE_PW = NBLK // NW
EXTRA = NBLK - BASE_PW * NW
NPAD = 10240
RPS = NPAD // NS

def _leaky(x):
    return jnp.where(x >= 0, x, 0.01 * x)


def _matT(a, w):
    return lax.dot_general(a, w, (((1,), (1,)), ((), ())),
                           preferred_element_type=jnp.float32)


def _sc_pass_body(idx_hbm, table_hbm, out_hbm,
                  acc_sh, rows0_v, rows1_v, ix0_v, ix1_v, zb_v, sem0, sem1):
    cid = lax.axis_index("c")
    sid = lax.axis_index("s")
    wid = cid * NS + sid
    zeros16 = jnp.zeros((16,), jnp.float32)

    def zb_row(r, c):
        for j in range(D // 16):
            zb_v[r, pl.ds(j * 16, 16)] = zeros16
        return c
    lax.fori_loop(0, 32, zb_row, 0)

    def zacc(i, c):
        pltpu.sync_copy(zb_v, acc_sh.at[pl.ds(sid * RPS + i * 32, 32)])
        return c
    lax.fori_loop(0, RPS // 32, zacc, 0)

    plsc.subcore_barrier()

    start = wid * BASE_PW + jnp.minimum(wid, EXTRA)
    nblk = BASE_PW + jnp.where(wid < EXTRA, 1, 0)

    def pair_body(p, c):
        b0 = start + 2 * p
        pltpu.sync_copy(idx_hbm.at[pl.ds(b0, 1)], ix0_v)
        g0 = pltpu.async_copy(table_hbm.at[ix0_v.at[0, 0]], rows0_v, sem0)
        pltpu.sync_copy(idx_hbm.at[pl.ds(b0 + 1, 1)], ix1_v)
        g1 = pltpu.async_copy(table_hbm.at[ix1_v.at[0, 0]], rows1_v, sem1)
        g0.wait()
        pltpu.sync_copy(rows0_v, acc_sh.at[ix0_v.at[0, 1]], add=True)
        g1.wait()
        pltpu.sync_copy(rows1_v, acc_sh.at[ix1_v.at[0, 1]], add=True)
        return c
    lax.fori_loop(0, nblk // 2, pair_body, 0)

    @pl.when(nblk % 2 == 1)
    def _tail():
        b = start + nblk - 1
        pltpu.sync_copy(idx_hbm.at[pl.ds(b, 1)], ix0_v)
        pltpu.async_copy(table_hbm.at[ix0_v.at[0, 0]], rows0_v, sem0).wait()
        pltpu.sync_copy(rows0_v, acc_sh.at[ix0_v.at[0, 1]], add=True)

    plsc.subcore_barrier()
    pltpu.sync_copy(acc_sh.at[pl.ds(sid * RPS, RPS)],
                    out_hbm.at[cid, pl.ds(sid * RPS, RPS)])


_sc_pass = pl.kernel(
    _sc_pass_body,
    out_type=jax.ShapeDtypeStruct((NC, NPAD, D), jnp.float32),
    mesh=plsc.VectorSubcoreMesh(core_axis_name="c", subcore_axis_name="s"),
    scratch_types=[
        pltpu.VMEM_SHARED((NPAD, D), jnp.float32),
        pltpu.VMEM((BLK, D), jnp.float32),
        pltpu.VMEM((BLK, D), jnp.float32),
        pltpu.VMEM((1, 2, BLK), jnp.int32),
        pltpu.VMEM((1, 2, BLK), jnp.int32),
        pltpu.VMEM((32, D), jnp.float32),
        pltpu.SemaphoreType.DMA,
        pltpu.SemaphoreType.DMA,
    ])


def _sc_count_body(gidx_hbm, sidx_hbm, out_hbm,
                   acc_sh, onesA_v, onesB_v, gi_v, si_v, zb_v, sem):
    cid = lax.axis_index("c")
    sid = lax.axis_index("s")
    wid = cid * NS + sid
    zeros16 = jnp.zeros((16,), jnp.float32)
    ones16 = jnp.ones((16,), jnp.float32)

    def ofill(r, c):
        for j in range(D // 16):
            onesA_v[r, pl.ds(j * 16, 16)] = ones16 if j == 0 else zeros16
            onesB_v[r, pl.ds(j * 16, 16)] = ones16 if j == 1 else zeros16
            zb_v[r % 32, pl.ds(j * 16, 16)] = zeros16
        return c
    lax.fori_loop(0, BLK, ofill, 0)

    def zacc(i, c):
        pltpu.sync_copy(zb_v, acc_sh.at[pl.ds(sid * RPS + i * 32, 32)])
        return c
    lax.fori_loop(0, RPS // 32, zacc, 0)

    plsc.subcore_barrier()

    start = wid * BASE_PW + jnp.minimum(wid, EXTRA)
    nblk = BASE_PW + jnp.where(wid < EXTRA, 1, 0)

    def blk_body(i, c):
        b = start + i
        pltpu.sync_copy(gidx_hbm.at[pl.ds(b, 1)], gi_v)
        pltpu.sync_copy(sidx_hbm.at[pl.ds(b, 1)], si_v)
        pltpu.sync_copy(onesA_v, acc_sh.at[gi_v.at[0]], add=True)
        pltpu.sync_copy(onesB_v, acc_sh.at[si_v.at[0]], add=True)
        return c
    lax.fori_loop(0, nblk, blk_body, 0)

    plsc.subcore_barrier()
    pltpu.sync_copy(acc_sh.at[pl.ds(sid * RPS, RPS)],
                    out_hbm.at[cid, pl.ds(sid * RPS, RPS)])


_sc_count = pl.kernel(
    _sc_count_body,
    out_type=jax.ShapeDtypeStruct((NC, NPAD, D), jnp.float32),
    mesh=plsc.VectorSubcoreMesh(core_axis_name="c", subcore_axis_name="s"),
    scratch_types=[
        pltpu.VMEM_SHARED((NPAD, D), jnp.float32),
        pltpu.VMEM((BLK, D), jnp.float32),
        pltpu.VMEM((BLK, D), jnp.float32),
        pltpu.VMEM((1, BLK), jnp.int32),
        pltpu.VMEM((1, BLK), jnp.int32),
        pltpu.VMEM((32, D), jnp.float32),
        pltpu.SemaphoreType.DMA,
    ])


MINBLK = 1280
MIN_GRID = E // MINBLK
NACC = 8


def _tc_min_body(col_ref, row_ref, x2_ref, out_ref, acc_v):
    step = pl.program_id(0)

    @pl.when(step == 0)
    def _init():
        acc_v[...] = jnp.full((NACC, M, D), jnp.inf, jnp.float32)

    for i in range(0, MINBLK, NACC):
        for k in range(NACC):
            c = col_ref[0, 0, i + k]
            r = row_ref[0, 0, i + k]
            acc_v[k, pl.ds(c, 1), :] = jnp.minimum(
                acc_v[k, pl.ds(c, 1), :], x2_ref[pl.ds(r, 1), :])

    @pl.when(step == MIN_GRID - 1)
    def _fin():
        red = acc_v[0]
        for k in range(1, NACC):
            red = jnp.minimum(red, acc_v[k])
        out_ref[...] = red


def _tc_min(col2, row2, x2):
    col3 = col2.reshape(MIN_GRID, 1, MINBLK)
    row3 = row2.reshape(MIN_GRID, 1, MINBLK)
    return pl.pallas_call(
        _tc_min_body,
        grid=(MIN_GRID,),
        in_specs=[
            pl.BlockSpec((1, 1, MINBLK), lambda i: (i, 0, 0),
                         memory_space=pltpu.SMEM),
            pl.BlockSpec((1, 1, MINBLK), lambda i: (i, 0, 0),
                         memory_space=pltpu.SMEM),
            pl.BlockSpec((N, D), lambda i: (0, 0)),
        ],
        out_specs=pl.BlockSpec((M, D), lambda i: (0, 0)),
        out_shape=jax.ShapeDtypeStruct((M, D), jnp.float32),
        scratch_shapes=[pltpu.VMEM((NACC, M, D), jnp.float32)],
    )(col3, row3, x2)


def _dense_pre_body(x_ref, xe_ref, ing, inb, ipW, ipb, eng, enb, epW, epb,
                    gn0w, gn0b, gn0ms, hgW, skW, skb,
                    gndw, gndb, gndms, hgdW, skdW, skdb,
                    xw_ref, skipx_ref, xwe_ref, skipxe_ref):
    eps = 1e-5
    x = x_ref[...]
    m = jnp.mean(x, axis=1, keepdims=True)
    v = jnp.mean((x - m) ** 2, axis=1, keepdims=True)
    xln = (x - m) / jnp.sqrt(v + eps) * ing[...] + inb[...]
    x1 = _leaky(_matT(xln, ipW[...]) + ipb[...])
    mu = jnp.mean(x1, axis=0, keepdims=True)
    out = x1 - mu * gn0ms[...]
    var = jnp.mean(out * out, axis=0, keepdims=True)
    xg = gn0w[...] * out / jnp.sqrt(var + eps) + gn0b[...]
    xw_ref[...] = _matT(xg, hgW[...])
    skipx_ref[...] = _matT(xg, skW[...]) + skb[...]
    xe = xe_ref[...]
    nrm = jnp.sqrt(jnp.sum(xe * xe, axis=1, keepdims=True))
    xe = xe / jnp.maximum(nrm, 1e-12)
    me = jnp.mean(xe, axis=1, keepdims=True)
    ve = jnp.mean((xe - me) ** 2, axis=1, keepdims=True)
    xeln = (xe - me) / jnp.sqrt(ve + eps) * eng[...] + enb[...]
    xe1 = _leaky(_matT(xeln, epW[...]) + epb[...])
    mue = jnp.mean(xe1, axis=0, keepdims=True)
    oute = xe1 - mue * gndms[...]
    vare = jnp.mean(oute * oute, axis=0, keepdims=True)
    xeg = gndw[...] * oute / jnp.sqrt(vare + eps) + gndb[...]
    xwe_ref[...] = _matT(xeg, hgdW[...])
    skipxe_ref[...] = _matT(xeg, skdW[...]) + skdb[...]


def _dense_pre(x, x_e, ing, inb, ipW, ipb, eng, enb, epW, epb,
               gn0w, gn0b, gn0ms, hgW, skW, skb,
               gndw, gndb, gndms, hgdW, skdW, skdb):
    outs = (jax.ShapeDtypeStruct((N, D), jnp.float32),) * 4
    return pl.pallas_call(
        _dense_pre_body,
        out_shape=outs,
    )(x, x_e, ing, inb, ipW, ipb, eng, enb, epW, epb,
      gn0w, gn0b, gn0ms, hgW, skW, skb,
      gndw, gndb, gndms, hgdW, skdW, skdb)


def _mid_body(p1, p3, cr, cc, oute_ref, oute2_ref,
              becol_ref, berow_ref):
    cnt_r = (cr[0] + cr[1]).reshape(1, N)
    cnt_c = (cc[0] + cc[1]).reshape(1, M)
    inv_r = jnp.where(cnt_r > 0, 1.0 / jnp.maximum(cnt_r, 1.0), 0.0)
    inv_c = jnp.where(cnt_c > 0, 1.0 / jnp.maximum(cnt_c, 1.0), 0.0)
    oute_ref[...] = (p1[0, :M, :] + p1[1, :M, :]) * inv_c.reshape(M, 1)
    oute2_ref[...] = (p3[0, :N, :] + p3[1, :N, :]) * inv_r.reshape(N, 1)
    becol_ref[...] = inv_c
    berow_ref[...] = inv_r


def _mid(p1, p3, cr, cc):
    outs = (jax.ShapeDtypeStruct((M, D), jnp.float32),
            jax.ShapeDtypeStruct((N, D), jnp.float32),
            jax.ShapeDtypeStruct((1, M), jnp.float32),
            jax.ShapeDtypeStruct((1, N), jnp.float32))
    return pl.pallas_call(_mid_body, out_shape=outs)(p1, p3, cr, cc)


def _x2_body(p2, dninv, hgb, skipx, x2_ref):
    raw = (p2[0, :N, :] + p2[1, :N, :]) * dninv[...].reshape(N, 1)
    x2_ref[...] = _leaky(raw + hgb[...]) + skipx[...]


def _x2(p2, dninv, hgb, skipx):
    return pl.pallas_call(
        _x2_body, out_shape=jax.ShapeDtypeStruct((N, D), jnp.float32))(
        p2, dninv, hgb, skipx)


def _final_body(p4, dninv_c, hgdb, skipxe, agg, fuseW, fuseb,
                linW, linb, out_ref):
    raw = (p4[0, :M, :] + p4[1, :M, :]) * dninv_c[...].reshape(M, 1)
    xe2 = _leaky(raw + hgdb[...]) + skipxe[...]
    fW = fuseW[...]
    h = (_matT(agg[...], fW[:, :D]) + _matT(xe2, fW[:, D:]) + fuseb[...])
    out_ref[...] = _matT(h, linW[...]) + linb[...]


def _final(p4, dninv_c, hgdb, skipxe, agg, fuseW, fuseb, linW, linb):
    return pl.pallas_call(
        _final_body, out_shape=jax.ShapeDtypeStruct((M, D), jnp.float32))(
        p4, dninv_c, hgdb, skipxe, agg, fuseW, fuseb, linW, linb)


def kernel(x, x_e, edge_index, in_norm_g, in_norm_b, in_proj_W, in_proj_b,
           e_norm_g, e_norm_b, e_proj_W, e_proj_b, gn0_w, gn0_b, gn0_ms,
           hg0_W, hg0_b, skip0_W, skip0_b, gnd0_w, gnd0_b, gnd0_ms,
           hgd0_W, hgd0_b, skipd0_W, skipd0_b, fuse_W, fuse_b, lin_W, lin_b):
    r2 = lambda a: a.reshape(1, D)
    row, col = edge_index[0], edge_index[1]
    row2 = row.reshape(NBLK, BLK)
    col2 = col.reshape(NBLK, BLK)
    rc2 = jnp.stack([row2, col2], axis=1)
    cr2 = jnp.stack([col2, row2], axis=1)

    xw, skip_x, xwe, skip_xe = _dense_pre(
        x, x_e, r2(in_norm_g), r2(in_norm_b), in_proj_W, r2(in_proj_b),
        r2(e_norm_g), r2(e_norm_b), e_proj_W, r2(e_proj_b),
        r2(gn0_w), r2(gn0_b), r2(gn0_ms), hg0_W, skip0_W, r2(skip0_b),
        r2(gnd0_w), r2(gnd0_b), r2(gnd0_ms), hgd0_W, skipd0_W, r2(skipd0_b))

    cntp = _sc_count(row2, col2)
    p1p = _sc_pass(rc2, xw)
    p3p = _sc_pass(cr2, xwe)
    out_e, out_e2, inv_c, inv_r = _mid(p1p, p3p,
                                       cntp[:, :N, 0], cntp[:, :M, 16])

    p2p = _sc_pass(cr2, out_e)
    x2 = _x2(p2p, inv_r, r2(hg0_b), skip_x)

    p4p = _sc_pass(rc2, out_e2)
    agg = _tc_min(col2, row2, x2)

    return _final(p4p, inv_c, r2(hgd0_b), skip_xe,
                  agg, fuse_W, r2(fuse_b), lin_W, r2(lin_b))


# speedup vs baseline: 6.3271x; 1.0195x over previous
import jax
import jax.numpy as jnp
from jax import lax
from jax.experimental import pallas as pl
from jax.experimental.pallas import tpu as pltpu
from jax.experimental.pallas import tpu_sc as plsc

N = 10000
M = 10000
E = 320000
D = 128

NC, NS = 2, 16
NW = NC * NS
BLK = 128
NBLK = E // BLK
BASE_PW = NBLK // NW
EXTRA = NBLK - BASE_PW * NW
NPAD = 10240
RPS = NPAD // NS

def _leaky(x):
    return jnp.where(x >= 0, x, 0.01 * x)


def _matT(a, w):
    return lax.dot_general(a, w, (((1,), (1,)), ((), ())),
                           preferred_element_type=jnp.float32)


def _sc_pass_body(idx_hbm, table_hbm, out_hbm,
                  acc_sh, rows0_v, rows1_v, ix0_v, ix1_v, zb_v, sem0, sem1):
    cid = lax.axis_index("c")
    sid = lax.axis_index("s")
    wid = cid * NS + sid
    zeros16 = jnp.zeros((16,), jnp.float32)

    def zb_row(r, c):
        for j in range(D // 16):
            zb_v[r, pl.ds(j * 16, 16)] = zeros16
        return c
    lax.fori_loop(0, 32, zb_row, 0)

    def zacc(i, c):
        pltpu.sync_copy(zb_v, acc_sh.at[pl.ds(sid * RPS + i * 32, 32)])
        return c
    lax.fori_loop(0, RPS // 32, zacc, 0)

    plsc.subcore_barrier()

    start = wid * BASE_PW + jnp.minimum(wid, EXTRA)
    nblk = BASE_PW + jnp.where(wid < EXTRA, 1, 0)

    def pair_body(p, c):
        b0 = start + 2 * p
        pltpu.sync_copy(idx_hbm.at[pl.ds(b0, 1)], ix0_v)
        g0 = pltpu.async_copy(table_hbm.at[ix0_v.at[0, 0]], rows0_v, sem0)
        pltpu.sync_copy(idx_hbm.at[pl.ds(b0 + 1, 1)], ix1_v)
        g1 = pltpu.async_copy(table_hbm.at[ix1_v.at[0, 0]], rows1_v, sem1)
        g0.wait()
        pltpu.sync_copy(rows0_v, acc_sh.at[ix0_v.at[0, 1]], add=True)
        g1.wait()
        pltpu.sync_copy(rows1_v, acc_sh.at[ix1_v.at[0, 1]], add=True)
        return c
    lax.fori_loop(0, nblk // 2, pair_body, 0)

    @pl.when(nblk % 2 == 1)
    def _tail():
        b = start + nblk - 1
        pltpu.sync_copy(idx_hbm.at[pl.ds(b, 1)], ix0_v)
        pltpu.async_copy(table_hbm.at[ix0_v.at[0, 0]], rows0_v, sem0).wait()
        pltpu.sync_copy(rows0_v, acc_sh.at[ix0_v.at[0, 1]], add=True)

    plsc.subcore_barrier()
    pltpu.sync_copy(acc_sh.at[pl.ds(sid * RPS, RPS)],
                    out_hbm.at[cid, pl.ds(sid * RPS, RPS)])


_sc_pass = pl.kernel(
    _sc_pass_body,
    out_type=jax.ShapeDtypeStruct((NC, NPAD, D), jnp.float32),
    mesh=plsc.VectorSubcoreMesh(core_axis_name="c", subcore_axis_name="s"),
    scratch_types=[
        pltpu.VMEM_SHARED((NPAD, D), jnp.float32),
        pltpu.VMEM((BLK, D), jnp.float32),
        pltpu.VMEM((BLK, D), jnp.float32),
        pltpu.VMEM((1, 2, BLK), jnp.int32),
        pltpu.VMEM((1, 2, BLK), jnp.int32),
        pltpu.VMEM((32, D), jnp.float32),
        pltpu.SemaphoreType.DMA,
        pltpu.SemaphoreType.DMA,
    ])


def _sc_count_body(gidx_hbm, sidx_hbm, out_hbm,
                   acc_sh, onesA_v, onesB_v, gi_v, si_v, zb_v, sem):
    cid = lax.axis_index("c")
    sid = lax.axis_index("s")
    wid = cid * NS + sid
    zeros16 = jnp.zeros((16,), jnp.float32)
    ones16 = jnp.ones((16,), jnp.float32)

    def ofill(r, c):
        for j in range(D // 16):
            onesA_v[r, pl.ds(j * 16, 16)] = ones16 if j == 0 else zeros16
            onesB_v[r, pl.ds(j * 16, 16)] = ones16 if j == 1 else zeros16
            zb_v[r % 32, pl.ds(j * 16, 16)] = zeros16
        return c
    lax.fori_loop(0, BLK, ofill, 0)

    def zacc(i, c):
        pltpu.sync_copy(zb_v, acc_sh.at[pl.ds(sid * RPS + i * 32, 32)])
        return c
    lax.fori_loop(0, RPS // 32, zacc, 0)

    plsc.subcore_barrier()

    start = wid * BASE_PW + jnp.minimum(wid, EXTRA)
    nblk = BASE_PW + jnp.where(wid < EXTRA, 1, 0)

    def blk_body(i, c):
        b = start + i
        pltpu.sync_copy(gidx_hbm.at[pl.ds(b, 1)], gi_v)
        pltpu.sync_copy(sidx_hbm.at[pl.ds(b, 1)], si_v)
        pltpu.sync_copy(onesA_v, acc_sh.at[gi_v.at[0]], add=True)
        pltpu.sync_copy(onesB_v, acc_sh.at[si_v.at[0]], add=True)
        return c
    lax.fori_loop(0, nblk, blk_body, 0)

    plsc.subcore_barrier()
    pltpu.sync_copy(acc_sh.at[pl.ds(sid * RPS, RPS)],
                    out_hbm.at[cid, pl.ds(sid * RPS, RPS)])


_sc_count = pl.kernel(
    _sc_count_body,
    out_type=jax.ShapeDtypeStruct((NC, NPAD, D), jnp.float32),
    mesh=plsc.VectorSubcoreMesh(core_axis_name="c", subcore_axis_name="s"),
    scratch_types=[
        pltpu.VMEM_SHARED((NPAD, D), jnp.float32),
        pltpu.VMEM((BLK, D), jnp.float32),
        pltpu.VMEM((BLK, D), jnp.float32),
        pltpu.VMEM((1, BLK), jnp.int32),
        pltpu.VMEM((1, BLK), jnp.int32),
        pltpu.VMEM((32, D), jnp.float32),
        pltpu.SemaphoreType.DMA,
    ])


MINBLK = 3200
MIN_GRID = E // MINBLK
NACC = 8


def _tc_min_body(col_ref, row_ref, x2_ref, out_ref, acc_v):
    step = pl.program_id(0)

    @pl.when(step == 0)
    def _init():
        acc_v[...] = jnp.full((NACC, M, D), jnp.inf, jnp.float32)

    for i in range(0, MINBLK, NACC):
        for k in range(NACC):
            c = col_ref[0, 0, i + k]
            r = row_ref[0, 0, i + k]
            acc_v[k, pl.ds(c, 1), :] = jnp.minimum(
                acc_v[k, pl.ds(c, 1), :], x2_ref[pl.ds(r, 1), :])

    @pl.when(step == MIN_GRID - 1)
    def _fin():
        red = acc_v[0]
        for k in range(1, NACC):
            red = jnp.minimum(red, acc_v[k])
        out_ref[...] = red


def _tc_min(col2, row2, x2):
    col3 = col2.reshape(MIN_GRID, 1, MINBLK)
    row3 = row2.reshape(MIN_GRID, 1, MINBLK)
    return pl.pallas_call(
        _tc_min_body,
        grid=(MIN_GRID,),
        in_specs=[
            pl.BlockSpec((1, 1, MINBLK), lambda i: (i, 0, 0),
                         memory_space=pltpu.SMEM),
            pl.BlockSpec((1, 1, MINBLK), lambda i: (i, 0, 0),
                         memory_space=pltpu.SMEM),
            pl.BlockSpec((N, D), lambda i: (0, 0)),
        ],
        out_specs=pl.BlockSpec((M, D), lambda i: (0, 0)),
        out_shape=jax.ShapeDtypeStruct((M, D), jnp.float32),
        scratch_shapes=[pltpu.VMEM((NACC, M, D), jnp.float32)],
    )(col3, row3, x2)


def _dense_pre_body(x_ref, xe_ref, ing, inb, ipW, ipb, eng, enb, epW, epb,
                    gn0w, gn0b, gn0ms, hgW, skW, skb,
                    gndw, gndb, gndms, hgdW, skdW, skdb,
                    xw_ref, skipx_ref, xwe_ref, skipxe_ref):
    eps = 1e-5
    x = x_ref[...]
    m = jnp.mean(x, axis=1, keepdims=True)
    v = jnp.mean((x - m) ** 2, axis=1, keepdims=True)
    xln = (x - m) / jnp.sqrt(v + eps) * ing[...] + inb[...]
    x1 = _leaky(_matT(xln, ipW[...]) + ipb[...])
    mu = jnp.mean(x1, axis=0, keepdims=True)
    out = x1 - mu * gn0ms[...]
    var = jnp.mean(out * out, axis=0, keepdims=True)
    xg = gn0w[...] * out / jnp.sqrt(var + eps) + gn0b[...]
    xw_ref[...] = _matT(xg, hgW[...])
    skipx_ref[...] = _matT(xg, skW[...]) + skb[...]
    xe = xe_ref[...]
    nrm = jnp.sqrt(jnp.sum(xe * xe, axis=1, keepdims=True))
    xe = xe / jnp.maximum(nrm, 1e-12)
    me = jnp.mean(xe, axis=1, keepdims=True)
    ve = jnp.mean((xe - me) ** 2, axis=1, keepdims=True)
    xeln = (xe - me) / jnp.sqrt(ve + eps) * eng[...] + enb[...]
    xe1 = _leaky(_matT(xeln, epW[...]) + epb[...])
    mue = jnp.mean(xe1, axis=0, keepdims=True)
    oute = xe1 - mue * gndms[...]
    vare = jnp.mean(oute * oute, axis=0, keepdims=True)
    xeg = gndw[...] * oute / jnp.sqrt(vare + eps) + gndb[...]
    xwe_ref[...] = _matT(xeg, hgdW[...])
    skipxe_ref[...] = _matT(xeg, skdW[...]) + skdb[...]


def _dense_pre(x, x_e, ing, inb, ipW, ipb, eng, enb, epW, epb,
               gn0w, gn0b, gn0ms, hgW, skW, skb,
               gndw, gndb, gndms, hgdW, skdW, skdb):
    outs = (jax.ShapeDtypeStruct((N, D), jnp.float32),) * 4
    return pl.pallas_call(
        _dense_pre_body,
        out_shape=outs,
    )(x, x_e, ing, inb, ipW, ipb, eng, enb, epW, epb,
      gn0w, gn0b, gn0ms, hgW, skW, skb,
      gndw, gndb, gndms, hgdW, skdW, skdb)


def _mid_body(p1, p3, cr, cc, oute_ref, oute2_ref,
              becol_ref, berow_ref):
    cnt_r = (cr[0] + cr[1]).reshape(1, N)
    cnt_c = (cc[0] + cc[1]).reshape(1, M)
    inv_r = jnp.where(cnt_r > 0, 1.0 / jnp.maximum(cnt_r, 1.0), 0.0)
    inv_c = jnp.where(cnt_c > 0, 1.0 / jnp.maximum(cnt_c, 1.0), 0.0)
    oute_ref[...] = (p1[0, :M, :] + p1[1, :M, :]) * inv_c.reshape(M, 1)
    oute2_ref[...] = (p3[0, :N, :] + p3[1, :N, :]) * inv_r.reshape(N, 1)
    becol_ref[...] = inv_c
    berow_ref[...] = inv_r


def _mid(p1, p3, cr, cc):
    outs = (jax.ShapeDtypeStruct((M, D), jnp.float32),
            jax.ShapeDtypeStruct((N, D), jnp.float32),
            jax.ShapeDtypeStruct((1, M), jnp.float32),
            jax.ShapeDtypeStruct((1, N), jnp.float32))
    return pl.pallas_call(_mid_body, out_shape=outs)(p1, p3, cr, cc)


def _x2_body(p2, dninv, hgb, skipx, x2_ref):
    raw = (p2[0, :N, :] + p2[1, :N, :]) * dninv[...].reshape(N, 1)
    x2_ref[...] = _leaky(raw + hgb[...]) + skipx[...]


def _x2(p2, dninv, hgb, skipx):
    return pl.pallas_call(
        _x2_body, out_shape=jax.ShapeDtypeStruct((N, D), jnp.float32))(
        p2, dninv, hgb, skipx)


def _final_body(p4, dninv_c, hgdb, skipxe, agg, fuseW, fuseb,
                linW, linb, out_ref):
    raw = (p4[0, :M, :] + p4[1, :M, :]) * dninv_c[...].reshape(M, 1)
    xe2 = _leaky(raw + hgdb[...]) + skipxe[...]
    fW = fuseW[...]
    h = (_matT(agg[...], fW[:, :D]) + _matT(xe2, fW[:, D:]) + fuseb[...])
    out_ref[...] = _matT(h, linW[...]) + linb[...]


def _final(p4, dninv_c, hgdb, skipxe, agg, fuseW, fuseb, linW, linb):
    return pl.pallas_call(
        _final_body, out_shape=jax.ShapeDtypeStruct((M, D), jnp.float32))(
        p4, dninv_c, hgdb, skipxe, agg, fuseW, fuseb, linW, linb)


def kernel(x, x_e, edge_index, in_norm_g, in_norm_b, in_proj_W, in_proj_b,
           e_norm_g, e_norm_b, e_proj_W, e_proj_b, gn0_w, gn0_b, gn0_ms,
           hg0_W, hg0_b, skip0_W, skip0_b, gnd0_w, gnd0_b, gnd0_ms,
           hgd0_W, hgd0_b, skipd0_W, skipd0_b, fuse_W, fuse_b, lin_W, lin_b):
    r2 = lambda a: a.reshape(1, D)
    row, col = edge_index[0], edge_index[1]
    row2 = row.reshape(NBLK, BLK)
    col2 = col.reshape(NBLK, BLK)
    rc2 = jnp.stack([row2, col2], axis=1)
    cr2 = jnp.stack([col2, row2], axis=1)

    xw, skip_x, xwe, skip_xe = _dense_pre(
        x, x_e, r2(in_norm_g), r2(in_norm_b), in_proj_W, r2(in_proj_b),
        r2(e_norm_g), r2(e_norm_b), e_proj_W, r2(e_proj_b),
        r2(gn0_w), r2(gn0_b), r2(gn0_ms), hg0_W, skip0_W, r2(skip0_b),
        r2(gnd0_w), r2(gnd0_b), r2(gnd0_ms), hgd0_W, skipd0_W, r2(skipd0_b))

    cntp = _sc_count(row2, col2)
    p1p = _sc_pass(rc2, xw)
    p3p = _sc_pass(cr2, xwe)
    out_e, out_e2, inv_c, inv_r = _mid(p1p, p3p,
                                       cntp[:, :N, 0], cntp[:, :M, 16])

    p2p = _sc_pass(cr2, out_e)
    x2 = _x2(p2p, inv_r, r2(hg0_b), skip_x)

    p4p = _sc_pass(rc2, out_e2)
    agg = _tc_min(col2, row2, x2)

    return _final(p4p, inv_c, r2(hgd0_b), skip_xe,
                  agg, fuse_W, r2(fuse_b), lin_W, r2(lin_b))
